# baseline (speedup 1.0000x reference)
import functools

import jax
import jax.numpy as jnp
from jax import lax
from jax.experimental import pallas as pl
from jax.experimental.pallas import tpu as pltpu
from jax.experimental.pallas import tpu_sc as plsc

_NPOOL = 64
_TOPK = 2
_DSCALE = 0.01
_DOT = functools.partial(
    jax.lax.dot_general,
    dimension_numbers=(((1,), (0,)), ((), ())),
    preferred_element_type=jnp.float32)


def _router_block(x_ref, W1_ref, b1_ref, W2_ref, b2_ref, temp_ref, Wp_ref,
                  bp_ref, gamma_ref, beta_ref, Wma_ref, Wmb_ref, bm_ref,
                  pool_ref, out_ref, wmat_ref):
    xb = x_ref[...]

    inter = jnp.maximum(_DOT(xb, W1_ref[...]) + b1_ref[...], 0.0)
    logits = _DOT(inter, W2_ref[...]) + b2_ref[...]

    temp = jnp.clip(temp_ref[0, 0], 0.1, 5.0)
    scaled = jnp.clip(logits / temp, -10.0, 10.0)

    col = jax.lax.broadcasted_iota(jnp.int32, scaled.shape, 1)
    m1 = jnp.max(scaled, axis=-1, keepdims=True)
    idx1 = jnp.min(jnp.where(scaled == m1, col, _NPOOL), axis=-1,
                   keepdims=True)
    masked = jnp.where(col == idx1, -jnp.inf, scaled)
    m2 = jnp.max(masked, axis=-1, keepdims=True)
    idx2 = jnp.min(jnp.where(masked == m2, col, _NPOOL), axis=-1,
                   keepdims=True)
    e2 = jnp.exp(m2 - m1)
    denom = 1.0 + e2
    w1 = 1.0 / denom
    w2 = e2 / denom
    wmat = (jnp.where(col == idx1, w1, 0.0)
            + jnp.where(col == idx2, w2, 0.0))
    wmat_ref[...] = wmat

    wmap = _DOT(wmat, pool_ref[...])

    px = _DOT(xb, Wp_ref[...]) + bp_ref[...]
    mu = jnp.mean(px, axis=-1, keepdims=True)
    var = jnp.mean((px - mu) ** 2, axis=-1, keepdims=True)
    ln = ((px - mu) / jnp.sqrt(var + 1e-5)) * gamma_ref[...] + beta_ref[...]

    out_ref[...] = (_DOT(ln, Wma_ref[...])
                    + _DOT(wmap, Wmb_ref[...]) + bm_ref[...])


def _sc_usage(wmat_hbm, part_hbm, wv, acc, *, nc):
    c = lax.axis_index("c")
    s = lax.axis_index("s")
    wid = s * nc + c
    rows = wv.shape[0]
    base = wid * rows
    pltpu.sync_copy(wmat_hbm.at[pl.ds(base, rows)], wv)
    for j in range(_NPOOL // 16):
        acc[pl.ds(j * 16, 16)] = jnp.zeros((16,), jnp.float32)

    @pl.loop(0, rows)
    def _(r):
        for j in range(_NPOOL // 16):
            sl = pl.ds(j * 16, 16)
            acc[sl] = acc[sl] + wv[r, sl]

    pltpu.sync_copy(acc, part_hbm.at[wid])


def _loss_block(part_ref, loss_ref, *, scale):
    u = jnp.sum(part_ref[...], axis=0, keepdims=True)
    uf = u / (jnp.sum(u) + 1e-8)
    d = uf - 1.0 / _NPOOL
    loss_ref[...] = (jnp.mean(d * d) * (scale * _DSCALE)).reshape(1, 1)


def kernel(x, tensor_pool, W1, b1, W2, b2, temperature, Wp, bp, gamma, beta,
           Wm, bm):
    B, S, H = x.shape
    M = B * S
    npool, tdim = tensor_pool.shape
    inter_dim = W1.shape[1]
    BM = 512
    grid = (M // BM,)
    scale = min(1.0, float(x.size) / (npool * _TOPK))

    xf = x.reshape(M, H)
    full = lambda shape: pl.BlockSpec(shape, lambda i: (0,) * len(shape))
    out, wmat = pl.pallas_call(
        _router_block,
        grid=grid,
        in_specs=[
            pl.BlockSpec((BM, H), lambda i: (i, 0)),
            full((H, inter_dim)),
            full((1, inter_dim)),
            full((inter_dim, npool)),
            full((1, npool)),
            full((1, 1)),
            full((H, tdim)),
            full((1, tdim)),
            full((1, tdim)),
            full((1, tdim)),
            pl.BlockSpec((tdim, tdim), lambda i: (0, 0)),
            pl.BlockSpec((tdim, tdim), lambda i: (1, 0)),
            full((1, tdim)),
            full((npool, tdim)),
        ],
        out_specs=(
            pl.BlockSpec((BM, tdim), lambda i: (i, 0)),
            pl.BlockSpec((BM, npool), lambda i: (i, 0)),
        ),
        out_shape=(
            jax.ShapeDtypeStruct((M, tdim), jnp.float32),
            jax.ShapeDtypeStruct((M, npool), jnp.float32),
        ),
    )(xf, W1, b1.reshape(1, -1), W2, b2.reshape(1, -1),
      temperature.reshape(1, 1), Wp, bp.reshape(1, -1),
      gamma.reshape(1, -1), beta.reshape(1, -1), Wm,
      Wm, bm.reshape(1, -1), tensor_pool)

    info = plsc.get_sparse_core_info()
    n_tiles = info.num_cores * info.num_subcores
    rows = M // n_tiles
    sc_usage = pl.kernel(
        functools.partial(_sc_usage, nc=info.num_cores),
        out_type=jax.ShapeDtypeStruct((n_tiles, npool), jnp.float32),
        mesh=plsc.VectorSubcoreMesh(core_axis_name="c", subcore_axis_name="s"),
        scratch_types=[pltpu.VMEM((rows, npool), jnp.float32),
                       pltpu.VMEM((npool,), jnp.float32)],
    )
    parts = sc_usage(wmat)

    loss = pl.pallas_call(
        functools.partial(_loss_block, scale=scale),
        out_shape=jax.ShapeDtypeStruct((1, 1), jnp.float32),
    )(parts)
    return out.reshape(B, S, tdim), loss[0, 0]


# speedup vs baseline: 1.2416x; 1.2416x over previous
import functools

import jax
import jax.numpy as jnp
from jax.experimental import pallas as pl
from jax.experimental.pallas import tpu as pltpu

_NPOOL = 64
_TOPK = 2
_DSCALE = 0.01
_DOT = functools.partial(
    jax.lax.dot_general,
    dimension_numbers=(((1,), (0,)), ((), ())),
    preferred_element_type=jnp.float32)


def _router_block(x_ref, W1_ref, b1_ref, W2_ref, b2_ref, temp_ref, Wp_ref,
                  bp_ref, gamma_ref, beta_ref, Wma_ref, Wmb_ref, bm_ref,
                  pool_ref, out_ref, loss_ref, usage_acc, *, scale):
    i = pl.program_id(0)
    n = pl.num_programs(0)
    xb = x_ref[...]

    inter = jnp.maximum(_DOT(xb, W1_ref[...]) + b1_ref[...], 0.0)
    logits = _DOT(inter, W2_ref[...]) + b2_ref[...]

    temp = jnp.clip(temp_ref[0, 0], 0.1, 5.0)
    scaled = jnp.clip(logits / temp, -10.0, 10.0)

    col = jax.lax.broadcasted_iota(jnp.int32, scaled.shape, 1)
    m1 = jnp.max(scaled, axis=-1, keepdims=True)
    idx1 = jnp.min(jnp.where(scaled == m1, col, _NPOOL), axis=-1,
                   keepdims=True)
    masked = jnp.where(col == idx1, -jnp.inf, scaled)
    m2 = jnp.max(masked, axis=-1, keepdims=True)
    idx2 = jnp.min(jnp.where(masked == m2, col, _NPOOL), axis=-1,
                   keepdims=True)
    e2 = jnp.exp(m2 - m1)
    denom = 1.0 + e2
    w1 = 1.0 / denom
    w2 = e2 / denom
    wmat = (jnp.where(col == idx1, w1, 0.0)
            + jnp.where(col == idx2, w2, 0.0))

    @pl.when(i == 0)
    def _():
        usage_acc[...] = jnp.zeros_like(usage_acc)
    usage_acc[...] += jnp.sum(wmat, axis=0, keepdims=True)

    wmap = _DOT(wmat, pool_ref[...])

    px = _DOT(xb, Wp_ref[...]) + bp_ref[...]
    mu = jnp.mean(px, axis=-1, keepdims=True)
    var = jnp.mean((px - mu) ** 2, axis=-1, keepdims=True)
    ln = ((px - mu) / jnp.sqrt(var + 1e-5)) * gamma_ref[...] + beta_ref[...]

    out = (_DOT(ln, Wma_ref[...])
           + _DOT(wmap, Wmb_ref[...]) + bm_ref[...])
    out_ref[...] = out

    @pl.when(i == n - 1)
    def _():
        u = usage_acc[...]
        uf = u / (jnp.sum(u) + 1e-8)
        d = uf - 1.0 / _NPOOL
        loss_ref[...] = (jnp.mean(d * d) * (scale * _DSCALE)).reshape(1, 1)


def kernel(x, tensor_pool, W1, b1, W2, b2, temperature, Wp, bp, gamma, beta,
           Wm, bm):
    B, S, H = x.shape
    M = B * S
    npool, tdim = tensor_pool.shape
    inter_dim = W1.shape[1]
    BM = 1024
    grid = (M // BM,)
    scale = min(1.0, float(x.size) / (npool * _TOPK))

    xf = x.reshape(M, H)
    full = lambda shape: pl.BlockSpec(shape, lambda i: (0,) * len(shape))
    out, loss = pl.pallas_call(
        functools.partial(_router_block, scale=scale),
        grid=grid,
        in_specs=[
            pl.BlockSpec((BM, H), lambda i: (i, 0)),
            full((H, inter_dim)),
            full((1, inter_dim)),
            full((inter_dim, npool)),
            full((1, npool)),
            full((1, 1)),
            full((H, tdim)),
            full((1, tdim)),
            full((1, tdim)),
            full((1, tdim)),
            pl.BlockSpec((tdim, tdim), lambda i: (0, 0)),
            pl.BlockSpec((tdim, tdim), lambda i: (1, 0)),
            full((1, tdim)),
            full((npool, tdim)),
        ],
        out_specs=(
            pl.BlockSpec((BM, tdim), lambda i: (i, 0)),
            pl.BlockSpec((1, 1), lambda i: (0, 0)),
        ),
        out_shape=(
            jax.ShapeDtypeStruct((M, tdim), jnp.float32),
            jax.ShapeDtypeStruct((1, 1), jnp.float32),
        ),
        scratch_shapes=[pltpu.VMEM((1, npool), jnp.float32)],
    )(xf, W1, b1.reshape(1, -1), W2, b2.reshape(1, -1),
      temperature.reshape(1, 1), Wp, bp.reshape(1, -1),
      gamma.reshape(1, -1), beta.reshape(1, -1), Wm,
      Wm, bm.reshape(1, -1), tensor_pool)
    return out.reshape(B, S, tdim), loss[0, 0]
